# Initial kernel scaffold; baseline (speedup 1.0000x reference)
import jax
import jax.numpy as jnp
from jax.experimental import pallas as pl


def kernel(x, edge_index, W1_l, W1_r, b1, W2_l, W2_r, b2):
    raise NotImplementedError("write your pallas kernel here")


# speedup vs baseline: 40.7009x; 40.7009x over previous
import functools

import jax
import jax.numpy as jnp
from jax import lax
from jax.experimental import pallas as pl
from jax.experimental.pallas import tpu as pltpu
from jax.experimental.pallas import tpu_sc as plsc

N_NODES = 100000
NC, NS = 2, 16
NW = NC * NS
K = 8
CHUNK = K * 128
ZROWS = 448
ROWS_PER_TILE = 14 * ZROWS
N_PAD = NS * ROWS_PER_TILE
F = 16

_mesh = plsc.VectorSubcoreMesh(
    core_axis_name="c", subcore_axis_name="s", num_cores=NC, num_subcores=NS)


def _make_sc_pass(n_chunks):
  @functools.partial(
      pl.kernel,
      out_type=jax.ShapeDtypeStruct((NC, N_PAD, F), jnp.float32),
      mesh=_mesh,
      compiler_params=pltpu.CompilerParams(use_tc_tiling_on_sc=False),
      scratch_types=[
          pltpu.VMEM_SHARED((N_PAD, F), jnp.float32),
          pltpu.VMEM((K, 128), jnp.int32),
          pltpu.VMEM((K, 128), jnp.int32),
          pltpu.VMEM((CHUNK, F), jnp.float32),
          pltpu.VMEM((ZROWS, F), jnp.float32),
          pltpu.SemaphoreType.DMA,
      ],
  )
  def sc_pass(src_hbm, dst_hbm, table_hbm, acc_out, acc_sh, sidx, didx, rows,
              zbuf, sem):
    c = lax.axis_index("c")
    s = lax.axis_index("s")

    zero16 = jnp.zeros((16,), jnp.float32)

    def zb(i, carry):
      zbuf[i, :] = zero16
      return carry

    lax.fori_loop(0, ZROWS, zb, 0)

    base_row = s * ROWS_PER_TILE

    def zc(r, carry):
      pltpu.sync_copy(zbuf, acc_sh.at[pl.ds(base_row + r * ZROWS, ZROWS)])
      return carry

    lax.fori_loop(0, ROWS_PER_TILE // ZROWS, zc, 0)
    plsc.subcore_barrier()

    w = c * NS + s

    def chunk(g, carry):
      row0 = (w * n_chunks + g) * K
      pltpu.sync_copy(src_hbm.at[pl.ds(row0, K)], sidx)
      pltpu.sync_copy(dst_hbm.at[pl.ds(row0, K)], didx)
      cps = [
          pltpu.async_copy(table_hbm.at[sidx.at[j]],
                           rows.at[pl.ds(j * 128, 128)], sem)
          for j in range(K)
      ]
      for cp in cps:
        cp.wait()
      for j in range(K):
        pltpu.sync_copy(rows.at[pl.ds(j * 128, 128)], acc_sh.at[didx.at[j]],
                        add=True)
      return carry

    lax.fori_loop(0, n_chunks, chunk, 0)
    plsc.subcore_barrier()

    pltpu.sync_copy(acc_sh.at[pl.ds(base_row, ROWS_PER_TILE)],
                    acc_out.at[c, pl.ds(base_row, ROWS_PER_TILE)])

  return sc_pass


_BLK1 = 2048


def _tc_mean(acc):
  def body(acc_ref, out_ref):
    a = acc_ref[0] + acc_ref[1]
    col = lax.broadcasted_iota(jnp.int32, (_BLK1, F), 1)
    cnt = jnp.sum(jnp.where(col == 8, a, 0.0), axis=1, keepdims=True)
    m = a / jnp.maximum(cnt, 1.0)
    out_ref[...] = jnp.where(col < 8, m, jnp.where(col == 8, 1.0, 0.0))

  return pl.pallas_call(
      body,
      grid=(N_PAD // _BLK1,),
      in_specs=[pl.BlockSpec((NC, _BLK1, F), lambda i: (0, i, 0))],
      out_specs=pl.BlockSpec((_BLK1, F), lambda i: (i, 0)),
      out_shape=jax.ShapeDtypeStruct((N_PAD, F), jnp.float32),
  )(acc)


_BLK2 = 2000


def _tc_final(x, m1aug, acc2, w1l_p, w1r_p, w1r, w2l, w2r, b2):
  def body(x_ref, m1_ref, acc_ref, w1lp_ref, w1rp_ref, w1r_ref, w2l_ref,
           w2r_ref, b2_ref, out_ref):
    a = acc_ref[0] + acc_ref[1]
    col = lax.broadcasted_iota(jnp.int32, (_BLK2, F), 1)
    cnt = jnp.sum(jnp.where(col == 8, a, 0.0), axis=1, keepdims=True)
    ma = a / jnp.maximum(cnt, 1.0)
    m1 = m1_ref[...]
    dot = functools.partial(jnp.dot, preferred_element_type=jnp.float32)
    ah = dot(ma, w1lp_ref[...]) + dot(m1, w1rp_ref[...])
    h = dot(m1, w1lp_ref[...]) + dot(x_ref[...], w1r_ref[...])
    out_ref[...] = dot(ah, w2l_ref[...]) + b2_ref[...] + dot(h, w2r_ref[...])

  return pl.pallas_call(
      body,
      grid=(N_NODES // _BLK2,),
      in_specs=[
          pl.BlockSpec((_BLK2, 8), lambda i: (i, 0)),
          pl.BlockSpec((_BLK2, F), lambda i: (i, 0)),
          pl.BlockSpec((NC, _BLK2, F), lambda i: (0, i, 0)),
          pl.BlockSpec((F, 16), lambda i: (0, 0)),
          pl.BlockSpec((F, 16), lambda i: (0, 0)),
          pl.BlockSpec((8, 16), lambda i: (0, 0)),
          pl.BlockSpec((16, 8), lambda i: (0, 0)),
          pl.BlockSpec((16, 8), lambda i: (0, 0)),
          pl.BlockSpec((1, 8), lambda i: (0, 0)),
      ],
      out_specs=pl.BlockSpec((_BLK2, 8), lambda i: (i, 0)),
      out_shape=jax.ShapeDtypeStruct((N_NODES, 8), jnp.float32),
  )(x, m1aug, acc2, w1l_p, w1r_p, w1r, w2l, w2r, b2)


def kernel(x, edge_index, W1_l, W1_r, b1, W2_l, W2_r, b2):
  ei = edge_index.astype(jnp.int32)
  src, dst = ei[0], ei[1]
  e = src.shape[0]
  n_chunks = -(-e // (NW * CHUNK))
  e_pad = NW * CHUNK * n_chunks
  pad = e_pad - e
  ar = jnp.arange(pad, dtype=jnp.int32)
  src_p = jnp.concatenate([src, ar % N_NODES]).reshape(e_pad // 128, 128)
  dst_p = jnp.concatenate(
      [dst, N_NODES + ar % (N_PAD - N_NODES)]).reshape(e_pad // 128, 128)

  n = x.shape[0]
  x_aug = jnp.concatenate(
      [x, jnp.ones((n, 1), x.dtype), jnp.zeros((n, 7), x.dtype)], axis=1)

  sc_pass = _make_sc_pass(n_chunks)
  acc1 = sc_pass(src_p, dst_p, x_aug)
  m1aug = _tc_mean(acc1)
  acc2 = sc_pass(src_p, dst_p, m1aug)

  w1l_p = jnp.concatenate([W1_l, b1[None, :], jnp.zeros((7, 16), jnp.float32)],
                          axis=0)
  w1r_p = jnp.concatenate([W1_r, jnp.zeros((8, 16), jnp.float32)], axis=0)
  return _tc_final(x, m1aug, acc2, w1l_p, w1r_p, W1_r, W2_l, W2_r,
                   b2.reshape(1, 8))


# speedup vs baseline: 41.9172x; 1.0299x over previous
import functools

import jax
import jax.numpy as jnp
from jax import lax
from jax.experimental import pallas as pl
from jax.experimental.pallas import tpu as pltpu
from jax.experimental.pallas import tpu_sc as plsc

N_NODES = 100000
NC, NS = 2, 16
NW = NC * NS
K = 4
CHUNK = K * 128
ZROWS = 448
ROWS_PER_TILE = 14 * ZROWS
N_PAD = NS * ROWS_PER_TILE
F = 16

_mesh = plsc.VectorSubcoreMesh(
    core_axis_name="c", subcore_axis_name="s", num_cores=NC, num_subcores=NS)


def _make_sc_pass(n_chunks):
  @functools.partial(
      pl.kernel,
      out_type=jax.ShapeDtypeStruct((NC, N_PAD, F), jnp.float32),
      mesh=_mesh,
      compiler_params=pltpu.CompilerParams(use_tc_tiling_on_sc=False),
      scratch_types=[
          pltpu.VMEM_SHARED((N_PAD, F), jnp.float32),
          pltpu.VMEM((K, 128), jnp.int32),
          pltpu.VMEM((K, 128), jnp.int32),
          pltpu.VMEM((K, 128), jnp.int32),
          pltpu.VMEM((K, 128), jnp.int32),
          pltpu.VMEM((CHUNK, F), jnp.float32),
          pltpu.VMEM((CHUNK, F), jnp.float32),
          pltpu.VMEM((ZROWS, F), jnp.float32),
          pltpu.SemaphoreType.DMA,
          pltpu.SemaphoreType.DMA,
      ],
  )
  def sc_pass(src_hbm, dst_hbm, table_hbm, acc_out, acc_sh, sidx0, didx0,
              sidx1, didx1, rows0, rows1, zbuf, sem_g, sem_s):
    c = lax.axis_index("c")
    s = lax.axis_index("s")

    zero16 = jnp.zeros((16,), jnp.float32)

    def zb(i, carry):
      zbuf[i, :] = zero16
      return carry

    lax.fori_loop(0, ZROWS, zb, 0)

    base_row = s * ROWS_PER_TILE

    def zc(r, carry):
      pltpu.sync_copy(zbuf, acc_sh.at[pl.ds(base_row + r * ZROWS, ZROWS)])
      return carry

    lax.fori_loop(0, ROWS_PER_TILE // ZROWS, zc, 0)
    plsc.subcore_barrier()

    w = c * NS + s

    def load_idx(g, sidx, didx):
      row0 = (w * n_chunks + g) * K
      pltpu.sync_copy(src_hbm.at[pl.ds(row0, K)], sidx)
      pltpu.sync_copy(dst_hbm.at[pl.ds(row0, K)], didx)

    def fire_gathers(sidx, rows):
      return [
          pltpu.async_copy(table_hbm.at[sidx.at[j]],
                           rows.at[pl.ds(j * 128, 128)], sem_g)
          for j in range(K)
      ]

    def fire_scatters(didx, rows):
      return [
          pltpu.async_copy(rows.at[pl.ds(j * 128, 128)],
                           acc_sh.at[didx.at[j]], sem_s, add=True)
          for j in range(K)
      ]

    def pair(i, carry):
      load_idx(2 * i, sidx0, didx0)
      ga = fire_gathers(sidx0, rows0)
      load_idx(2 * i + 1, sidx1, didx1)
      for cp in ga:
        cp.wait()
      sa = fire_scatters(didx0, rows0)
      gb = fire_gathers(sidx1, rows1)
      for cp in gb:
        cp.wait()
      sb = fire_scatters(didx1, rows1)
      for cp in sa:
        cp.wait()
      for cp in sb:
        cp.wait()
      return carry

    lax.fori_loop(0, n_chunks // 2, pair, 0)
    plsc.subcore_barrier()

    pltpu.sync_copy(acc_sh.at[pl.ds(base_row, ROWS_PER_TILE)],
                    acc_out.at[c, pl.ds(base_row, ROWS_PER_TILE)])

  return sc_pass


_BLK1 = 2048


def _tc_mean(acc):
  def body(acc_ref, out_ref):
    a = acc_ref[0] + acc_ref[1]
    col = lax.broadcasted_iota(jnp.int32, (_BLK1, F), 1)
    cnt = jnp.sum(jnp.where(col == 8, a, 0.0), axis=1, keepdims=True)
    m = a / jnp.maximum(cnt, 1.0)
    out_ref[...] = jnp.where(col < 8, m, jnp.where(col == 8, 1.0, 0.0))

  return pl.pallas_call(
      body,
      grid=(N_PAD // _BLK1,),
      in_specs=[pl.BlockSpec((NC, _BLK1, F), lambda i: (0, i, 0))],
      out_specs=pl.BlockSpec((_BLK1, F), lambda i: (i, 0)),
      out_shape=jax.ShapeDtypeStruct((N_PAD, F), jnp.float32),
  )(acc)


_BLK2 = 2000


def _tc_final(x, m1aug, acc2, w1l_p, w1r_p, w1r, w2l, w2r, b2):
  def body(x_ref, m1_ref, acc_ref, w1lp_ref, w1rp_ref, w1r_ref, w2l_ref,
           w2r_ref, b2_ref, out_ref):
    a = acc_ref[0] + acc_ref[1]
    col = lax.broadcasted_iota(jnp.int32, (_BLK2, F), 1)
    cnt = jnp.sum(jnp.where(col == 8, a, 0.0), axis=1, keepdims=True)
    ma = a / jnp.maximum(cnt, 1.0)
    m1 = m1_ref[...]
    dot = functools.partial(jnp.dot, preferred_element_type=jnp.float32)
    ah = dot(ma, w1lp_ref[...]) + dot(m1, w1rp_ref[...])
    h = dot(m1, w1lp_ref[...]) + dot(x_ref[...], w1r_ref[...])
    out_ref[...] = dot(ah, w2l_ref[...]) + b2_ref[...] + dot(h, w2r_ref[...])

  return pl.pallas_call(
      body,
      grid=(N_NODES // _BLK2,),
      in_specs=[
          pl.BlockSpec((_BLK2, 8), lambda i: (i, 0)),
          pl.BlockSpec((_BLK2, F), lambda i: (i, 0)),
          pl.BlockSpec((NC, _BLK2, F), lambda i: (0, i, 0)),
          pl.BlockSpec((F, 16), lambda i: (0, 0)),
          pl.BlockSpec((F, 16), lambda i: (0, 0)),
          pl.BlockSpec((8, 16), lambda i: (0, 0)),
          pl.BlockSpec((16, 8), lambda i: (0, 0)),
          pl.BlockSpec((16, 8), lambda i: (0, 0)),
          pl.BlockSpec((1, 8), lambda i: (0, 0)),
      ],
      out_specs=pl.BlockSpec((_BLK2, 8), lambda i: (i, 0)),
      out_shape=jax.ShapeDtypeStruct((N_NODES, 8), jnp.float32),
  )(x, m1aug, acc2, w1l_p, w1r_p, w1r, w2l, w2r, b2)


def kernel(x, edge_index, W1_l, W1_r, b1, W2_l, W2_r, b2):
  ei = edge_index.astype(jnp.int32)
  src, dst = ei[0], ei[1]
  e = src.shape[0]
  n_chunks = 2 * -(-e // (NW * CHUNK * 2))
  e_pad = NW * CHUNK * n_chunks
  pad = e_pad - e
  ar = jnp.arange(pad, dtype=jnp.int32)
  src_p = jnp.concatenate([src, ar % N_NODES]).reshape(e_pad // 128, 128)
  dst_p = jnp.concatenate(
      [dst, N_NODES + ar % (N_PAD - N_NODES)]).reshape(e_pad // 128, 128)

  n = x.shape[0]
  x_aug = jnp.concatenate(
      [x, jnp.ones((n, 1), x.dtype), jnp.zeros((n, 7), x.dtype)], axis=1)

  sc_pass = _make_sc_pass(n_chunks)
  acc1 = sc_pass(src_p, dst_p, x_aug)
  m1aug = _tc_mean(acc1)
  acc2 = sc_pass(src_p, dst_p, m1aug)

  w1l_p = jnp.concatenate([W1_l, b1[None, :], jnp.zeros((7, 16), jnp.float32)],
                          axis=0)
  w1r_p = jnp.concatenate([W1_r, jnp.zeros((8, 16), jnp.float32)], axis=0)
  return _tc_final(x, m1aug, acc2, w1l_p, w1r_p, W1_r, W2_l, W2_r,
                   b2.reshape(1, 8))
